# baseline (speedup 1.0000x reference)
import jax
import jax.numpy as jnp
from jax.experimental import pallas as pl
from jax.experimental.pallas import tpu as pltpu

L = 128
H = 512
BSZ = 16
N = 20
TOPK = 5


def _scores_kernel(q_ref, k_ref, out_ref):
    q = q_ref[...]
    k = k_ref[...].reshape(N * L, H)
    att = jax.lax.dot_general(
        k, q, (((1,), (1,)), ((), ())),
        preferred_element_type=jnp.float32)
    slot = jax.lax.broadcasted_iota(jnp.int32, (1, 1, N), 2)
    acc = jnp.full((1, 1, N), -jnp.inf, dtype=jnp.float32)
    for n in range(N):
        acc = jnp.where(slot == n, jnp.max(att[n * L:(n + 1) * L, :]), acc)
    out_ref[...] = acc


def _topk_kernel(s_ref, out_ref):
    s = s_ref[...]
    col = jax.lax.broadcasted_iota(jnp.int32, (BSZ, N), 1)
    for k in range(TOPK):
        m = jnp.max(s, axis=1, keepdims=True)
        hit = jnp.where(s == m, col, N)
        idx = jnp.min(hit, axis=1, keepdims=True)
        out_ref[:, k:k + 1] = idx.astype(jnp.int32)
        s = jnp.where(col == idx, -jnp.inf, s)


def kernel(query, keys, values):
    del values
    q2 = query.reshape(L, BSZ * H)
    keys4 = keys.reshape(N, BSZ, L, H)

    scores = pl.pallas_call(
        _scores_kernel,
        grid=(BSZ,),
        in_specs=[
            pl.BlockSpec((L, H), lambda b: (0, b)),
            pl.BlockSpec((N, 1, L, H), lambda b: (0, b, 0, 0)),
        ],
        out_specs=pl.BlockSpec((1, 1, N), lambda b: (b, 0, 0)),
        out_shape=jax.ShapeDtypeStruct((BSZ, 1, N), jnp.float32),
    )(q2, keys4)

    s2 = scores.reshape(BSZ, N)
    topk_bk = pl.pallas_call(
        _topk_kernel,
        in_specs=[pl.BlockSpec((BSZ, N), lambda: (0, 0))],
        out_specs=pl.BlockSpec((BSZ, TOPK), lambda: (0, 0)),
        out_shape=jax.ShapeDtypeStruct((BSZ, TOPK), jnp.int32),
    )(s2)

    return (scores, topk_bk.T)


# speedup vs baseline: 2.1800x; 2.1800x over previous
import jax
import jax.numpy as jnp
from jax.experimental import pallas as pl
from jax.experimental.pallas import tpu as pltpu

L = 128
H = 512
BSZ = 16
N = 20
TOPK = 5


def _scores_kernel(q_ref, k_ref, out_ref):
    n = pl.program_id(0)

    @pl.when(n == 0)
    def _():
        out_ref[...] = jnp.full((BSZ, N), -jnp.inf, dtype=jnp.float32)

    k3 = k_ref[0].reshape(BSZ, L, H)
    batch = jax.lax.broadcasted_iota(jnp.int32, (BSZ, 1), 0)
    acc = jnp.full((BSZ, 1), -jnp.inf, dtype=jnp.float32)
    for b in range(BSZ):
        att = jax.lax.dot_general(
            k3[b], q_ref[:, b, :], (((1,), (1,)), ((), ())),
            preferred_element_type=jnp.float32)
        acc = jnp.where(batch == b, jnp.max(att), acc)
    slot = jax.lax.broadcasted_iota(jnp.int32, (BSZ, N), 1)
    out_ref[...] = jnp.where(slot == n, acc, out_ref[...])


def _topk_kernel(s_ref, out_ref):
    s = s_ref[...]
    col = jax.lax.broadcasted_iota(jnp.int32, (BSZ, N), 1)
    for k in range(TOPK):
        m = jnp.max(s, axis=1, keepdims=True)
        hit = jnp.where(s == m, col, N)
        idx = jnp.min(hit, axis=1, keepdims=True)
        out_ref[:, k:k + 1] = idx.astype(jnp.int32)
        s = jnp.where(col == idx, -jnp.inf, s)


def kernel(query, keys, values):
    del values
    q3 = query.reshape(L, BSZ, H)

    scores = pl.pallas_call(
        _scores_kernel,
        grid=(N,),
        in_specs=[
            pl.BlockSpec((L, BSZ, H), lambda n: (0, 0, 0)),
            pl.BlockSpec((1, BSZ, L * H), lambda n: (n, 0, 0)),
        ],
        out_specs=pl.BlockSpec((BSZ, N), lambda n: (0, 0)),
        out_shape=jax.ShapeDtypeStruct((BSZ, N), jnp.float32),
    )(q3, keys)

    topk_bk = pl.pallas_call(
        _topk_kernel,
        in_specs=[pl.BlockSpec((BSZ, N), lambda: (0, 0))],
        out_specs=pl.BlockSpec((BSZ, TOPK), lambda: (0, 0)),
        out_shape=jax.ShapeDtypeStruct((BSZ, TOPK), jnp.int32),
    )(scores)

    return (scores.reshape(BSZ, 1, N), topk_bk.T)
